# Initial kernel scaffold; baseline (speedup 1.0000x reference)
import jax
import jax.numpy as jnp
from jax.experimental import pallas as pl


def kernel(x, edge_index, edge_attr, w1_msg, b1_msg, w1_edge, b1_edge, a1, w1_self, b1_self, w2_msg, b2_msg, w2_edge, b2_edge, a2, w2_self, b2_self, w3_msg, b3_msg, w3_edge, b3_edge, a3, w3_self, b3_self, w4_msg, b4_msg, a4, w4_self, b4_self):
    raise NotImplementedError("write your pallas kernel here")


# speedup vs baseline: 12.1599x; 12.1599x over previous
import functools

import numpy as np
import jax
import jax.numpy as jnp
from jax import lax
from jax.experimental import pallas as pl
from jax.experimental.pallas import tpu as pltpu
from jax.experimental.pallas import tpu_sc as plsc

N = 10000
NP = 10240
E = 80000
D_EDGE = 10
HP = 16
NC, NS = 2, 16
NW = NC * NS
NPS = NP // NS

_SPECS = [(1035, 128, 8, True), (128, 64, 8, True), (64, 32, 16, True), (32, 2, 1, False)]

f32 = jnp.float32
i32 = jnp.int32


def _prep_call(x, w_msg, bm, w_self, bs, am, bn):
    n, cin = x.shape
    hc = w_msg.shape[1]
    cout = w_self.shape[1]

    def body(x_ref, wm_ref, bm_ref, ws_ref, bs_ref, am_ref, xm_ref, xs_ref, axm_ref):
        xb = x_ref[...]
        xm = jnp.dot(xb, wm_ref[...], preferred_element_type=f32) + bm_ref[...]
        xm_ref[...] = xm
        xs_ref[...] = jnp.dot(xb, ws_ref[...], preferred_element_type=f32) + bs_ref[...]
        axm_ref[...] = jnp.dot(xm, am_ref[...], preferred_element_type=f32)

    grid = (n // bn,)
    return pl.pallas_call(
        body,
        grid=grid,
        in_specs=[
            pl.BlockSpec((bn, cin), lambda i: (i, 0)),
            pl.BlockSpec((cin, hc), lambda i: (0, 0)),
            pl.BlockSpec((1, hc), lambda i: (0, 0)),
            pl.BlockSpec((cin, cout), lambda i: (0, 0)),
            pl.BlockSpec((1, cout), lambda i: (0, 0)),
            pl.BlockSpec((hc, HP), lambda i: (0, 0)),
        ],
        out_specs=[
            pl.BlockSpec((bn, hc), lambda i: (i, 0)),
            pl.BlockSpec((bn, cout), lambda i: (i, 0)),
            pl.BlockSpec((bn, HP), lambda i: (i, 0)),
        ],
        out_shape=[
            jax.ShapeDtypeStruct((n, hc), f32),
            jax.ShapeDtypeStruct((n, cout), f32),
            jax.ShapeDtypeStruct((n, HP), f32),
        ],
    )(x, w_msg, bm.reshape(1, -1), w_self, bs.reshape(1, -1), am)


def _mm_call(a, b, bm):
    m, k = a.shape
    ncol = b.shape[1]
    bm_rows = bm

    def body(a_ref, b_ref, o_ref):
        o_ref[...] = jnp.dot(a_ref[...], b_ref[...], preferred_element_type=f32)

    return pl.pallas_call(
        body,
        grid=(m // bm_rows,),
        in_specs=[
            pl.BlockSpec((bm_rows, k), lambda i: (i, 0)),
            pl.BlockSpec((k, ncol), lambda i: (0, 0)),
        ],
        out_specs=pl.BlockSpec((bm_rows, ncol), lambda i: (i, 0)),
        out_shape=jax.ShapeDtypeStruct((m, ncol), f32),
    )(a, b)


def _post_edge_call(acc2, s2, wp, xs, bn, apply_elu):
    n, cout = xs.shape
    sw = wp.shape[0]

    def body(a0_ref, a1_ref, s0_ref, s1_ref, wp_ref, xs_ref, o_ref):
        s = s0_ref[...] + s1_ref[...]
        o = a0_ref[...] + a1_ref[...] + xs_ref[...] + jnp.dot(
            s, wp_ref[...], preferred_element_type=f32)
        if apply_elu:
            o = jnp.where(o > 0, o, jnp.exp(jnp.minimum(o, 0.0)) - 1.0)
        o_ref[...] = o

    return pl.pallas_call(
        body,
        grid=(n // bn,),
        in_specs=[
            pl.BlockSpec((bn, cout), lambda i: (i, 0)),
            pl.BlockSpec((bn, cout), lambda i: (i, 0)),
            pl.BlockSpec((bn, sw), lambda i: (i, 0)),
            pl.BlockSpec((bn, sw), lambda i: (i, 0)),
            pl.BlockSpec((sw, cout), lambda i: (0, 0)),
            pl.BlockSpec((bn, cout), lambda i: (i, 0)),
        ],
        out_specs=pl.BlockSpec((bn, cout), lambda i: (i, 0)),
        out_shape=jax.ShapeDtypeStruct((n, cout), f32),
    )(acc2[0], acc2[1], s2[0], s2[1], wp, xs)


def _post_plain_call(acc2, xs, bn):
    n, cout = xs.shape
    cp = acc2.shape[2]

    def body(a0_ref, a1_ref, xs_ref, o_ref):
        o_ref[...] = (a0_ref[...] + a1_ref[...])[:, :cout] + xs_ref[...]

    return pl.pallas_call(
        body,
        grid=(n // bn,),
        in_specs=[
            pl.BlockSpec((bn, cp), lambda i: (i, 0)),
            pl.BlockSpec((bn, cp), lambda i: (i, 0)),
            pl.BlockSpec((bn, cout), lambda i: (i, 0)),
        ],
        out_specs=pl.BlockSpec((bn, cout), lambda i: (i, 0)),
        out_shape=jax.ShapeDtypeStruct((n, cout), f32),
    )(acc2[0], acc2[1], xs)


_GDN = lax.GatherDimensionNumbers(
    offset_dims=(), collapsed_slice_dims=(0,), start_index_map=(0,))


def _vperm(v, idx):
    return lax.gather(v, idx[:, None], _GDN, (1,),
                      mode=lax.GatherScatterMode.PROMISE_IN_BOUNDS)


def _splat(v, k):
    return _vperm(v, jnp.full((HP,), k, i32))


def _sc_alpha_call(src, dst, axm, aem, has_edge):
    se = 128
    nch = E // se
    mesh = plsc.VectorSubcoreMesh(
        core_axis_name="c", subcore_axis_name="s", num_cores=NC, num_subcores=NS)
    zeros = jnp.zeros((NP, HP), f32)

    scratch = [
        pltpu.VMEM((se,), i32),
        pltpu.VMEM((se,), i32),
        pltpu.VMEM((se, HP), f32),
        pltpu.VMEM((se, HP), f32),
        pltpu.VMEM((se, HP), f32),
        pltpu.VMEM_SHARED((NP, HP), f32),
        pltpu.SemaphoreType.DMA,
    ]

    def body(src_h, dst_h, axm_h, aem_h, zero_h, ex_h, d2_h,
             srcb, dstb, axg, aemb, exb, dsh, sem):
        c = lax.axis_index("c")
        s = lax.axis_index("s")
        wid = c * NS + s
        pltpu.sync_copy(zero_h.at[pl.ds(s * NPS, NPS)], dsh.at[pl.ds(s * NPS, NPS)])
        plsc.subcore_barrier()

        def chunk(t, _):
            j = wid + t * NW
            base = j * se
            pltpu.sync_copy(src_h.at[pl.ds(base, se)], srcb)
            pltpu.sync_copy(dst_h.at[pl.ds(base, se)], dstb)
            pltpu.async_copy(axm_h.at[srcb], axg, sem).wait()
            if has_edge:
                pltpu.sync_copy(aem_h.at[pl.ds(base, se)], aemb)

            def edge(i, _):
                a = axg[i]
                if has_edge:
                    a = a + aemb[i]
                alo = jnp.where(a > 0, a, 0.2 * a)
                exb[i] = jnp.exp(alo)
                return 0

            lax.fori_loop(0, se, edge, 0)
            pltpu.sync_copy(exb, ex_h.at[pl.ds(base, se)])
            pltpu.sync_copy(exb, dsh.at[dstb], add=True)
            return 0

        nmine = (nch - wid + NW - 1) // NW
        lax.fori_loop(0, nmine, chunk, 0)
        plsc.subcore_barrier()
        pltpu.sync_copy(dsh.at[pl.ds(s * NPS, NPS)],
                        d2_h.at[c, pl.ds(s * NPS, NPS)])

    k = pl.kernel(
        body,
        out_type=[
            jax.ShapeDtypeStruct((E, HP), f32),
            jax.ShapeDtypeStruct((NC, NP, HP), f32),
        ],
        mesh=mesh,
        scratch_types=scratch,
        compiler_params=pltpu.CompilerParams(use_tc_tiling_on_sc=False),
    )
    if aem is None:
        aem = jnp.zeros((1, HP), f32)
    return k(src, dst, axm, aem, zeros)


def _sc_out1_call(src, dst, ex, da, db, xm, h, cout, se):
    hc = xm.shape[1]
    nch = E // se
    inv_h = 1.0 / h
    nr = cout // HP if cout >= HP else 1
    cp = max(cout, HP)
    mesh = plsc.VectorSubcoreMesh(
        core_axis_name="c", subcore_axis_name="s", num_cores=NC, num_subcores=NS)
    zacc = jnp.zeros((NP, cp), f32)

    scratch = [
        pltpu.VMEM((se,), i32),
        pltpu.VMEM((se,), i32),
        pltpu.VMEM((se, HP), f32),
        pltpu.VMEM((se, HP), f32),
        pltpu.VMEM((se, HP), f32),
        pltpu.VMEM((se, hc), f32),
        pltpu.VMEM((se, cp), f32),
        pltpu.VMEM_SHARED((NP, cp), f32),
        pltpu.SemaphoreType.DMA,
    ]

    def body(src_h, dst_h, ex_h, da_h, db_h, xm_h, zacc_h, acc2_h,
             srcb, dstb, exb, dab, dbb, xmg, rb, accsh, sem):
        c = lax.axis_index("c")
        s = lax.axis_index("s")
        wid = c * NS + s
        pltpu.sync_copy(zacc_h.at[pl.ds(s * NPS, NPS)], accsh.at[pl.ds(s * NPS, NPS)])
        plsc.subcore_barrier()

        def chunk(t, _):
            j = wid + t * NW
            base = j * se
            pltpu.sync_copy(src_h.at[pl.ds(base, se)], srcb)
            pltpu.sync_copy(dst_h.at[pl.ds(base, se)], dstb)
            pltpu.async_copy(xm_h.at[srcb], xmg, sem).wait()
            pltpu.sync_copy(ex_h.at[pl.ds(base, se)], exb)
            pltpu.async_copy(da_h.at[dstb], dab, sem).wait()
            pltpu.async_copy(db_h.at[dstb], dbb, sem).wait()
            iota = lax.iota(i32, HP)

            def edge(i, _):
                dv = dab[i] + dbb[i]
                beta = exb[i] / (dv + 1e-16) * inv_h
                bks = [_vperm(beta, iota * 0 + k2) for k2 in range(h)]
                for t2 in range(nr):
                    acc = bks[0] * xmg[i, pl.ds(t2 * HP, HP)]
                    for k2 in range(1, h):
                        acc = acc + bks[k2] * xmg[i, pl.ds(k2 * cout + t2 * HP, HP)]
                    rb[i, pl.ds(t2 * HP, HP)] = acc
                return 0

            lax.fori_loop(0, se, edge, 0)
            pltpu.sync_copy(rb, accsh.at[dstb], add=True)
            return 0

        nmine = (nch - wid + NW - 1) // NW
        lax.fori_loop(0, nmine, chunk, 0)
        plsc.subcore_barrier()
        pltpu.sync_copy(accsh.at[pl.ds(s * NPS, NPS)],
                        acc2_h.at[c, pl.ds(s * NPS, NPS)])

    k = pl.kernel(
        body,
        out_type=[
            jax.ShapeDtypeStruct((NC, NP, cp), f32),
        ],
        mesh=mesh,
        scratch_types=scratch,
        compiler_params=pltpu.CompilerParams(use_tc_tiling_on_sc=False),
    )
    return k(src, dst, ex, da, db, xm, zacc)[0]


def _sc_sedge_call(src, dst, ex, da, db, ea16, h, se):
    sw = h * D_EDGE
    nch = E // se
    inv_h = 1.0 / h
    nsreg = sw // HP
    mesh = plsc.VectorSubcoreMesh(
        core_axis_name="c", subcore_axis_name="s", num_cores=NC, num_subcores=NS)
    zs = jnp.zeros((NP, sw), f32)

    scratch = [
        pltpu.VMEM((se,), i32),
        pltpu.VMEM((se,), i32),
        pltpu.VMEM((se, HP), f32),
        pltpu.VMEM((se, HP), f32),
        pltpu.VMEM((se, HP), f32),
        pltpu.VMEM((se, HP), f32),
        pltpu.VMEM((se, sw), f32),
        pltpu.VMEM_SHARED((NP, sw), f32),
        pltpu.SemaphoreType.DMA,
    ]

    def body(src_h, dst_h, ex_h, da_h, db_h, ea_h, zs_h, s2_h,
             srcb, dstb, exb, dab, dbb, eab, sb, ssh, sem):
        c = lax.axis_index("c")
        s = lax.axis_index("s")
        wid = c * NS + s
        pltpu.sync_copy(zs_h.at[pl.ds(s * NPS, NPS)], ssh.at[pl.ds(s * NPS, NPS)])
        plsc.subcore_barrier()

        def chunk(t, _):
            j = wid + t * NW
            base = j * se
            pltpu.sync_copy(dst_h.at[pl.ds(base, se)], dstb)
            pltpu.sync_copy(ex_h.at[pl.ds(base, se)], exb)
            pltpu.async_copy(da_h.at[dstb], dab, sem).wait()
            pltpu.async_copy(db_h.at[dstb], dbb, sem).wait()
            pltpu.sync_copy(ea_h.at[pl.ds(base, se)], eab)
            iota = lax.iota(i32, HP)

            def edge(i, _):
                dv = dab[i] + dbb[i]
                beta = exb[i] / (dv + 1e-16) * inv_h
                eav = eab[i]
                for j2 in range(nsreg):
                    p = iota + j2 * HP
                    hd = lax.shift_right_logical(p * 205, 11)
                    sreg = _vperm(beta, hd) * _vperm(eav, p - hd * D_EDGE)
                    sb[i, pl.ds(j2 * HP, HP)] = sreg
                return 0

            lax.fori_loop(0, se, edge, 0)
            pltpu.sync_copy(sb, ssh.at[dstb], add=True)
            return 0

        nmine = (nch - wid + NW - 1) // NW
        lax.fori_loop(0, nmine, chunk, 0)
        plsc.subcore_barrier()
        pltpu.sync_copy(ssh.at[pl.ds(s * NPS, NPS)],
                        s2_h.at[c, pl.ds(s * NPS, NPS)])

    k = pl.kernel(
        body,
        out_type=[
            jax.ShapeDtypeStruct((NC, NP, sw), f32),
        ],
        mesh=mesh,
        scratch_types=scratch,
        compiler_params=pltpu.CompilerParams(use_tc_tiling_on_sc=False),
    )
    return k(src, dst, ex, da, db, ea16, zs)[0]


def _layer(x, src, dst, ea16, aem, w_msg, b_msg, w_edge, b_edge, att,
           w_self, b_self, h, cout, last):
    cin = x.shape[1]
    hc = h * cout
    has_edge = w_edge is not None

    eye = jnp.eye(HP, dtype=f32)[:h]
    am = (att[0][:, :, None] * eye[:, None, :]).reshape(hc, HP)

    bm = b_msg + b_edge if has_edge else b_msg
    if hc < HP:
        w_msg_p = jnp.pad(w_msg, ((0, 0), (0, HP - hc)))
        bm_p = jnp.pad(bm, (0, HP - hc))
        am_p = jnp.pad(am, ((0, HP - hc), (0, 0)))
    else:
        w_msg_p, bm_p, am_p = w_msg, bm, am

    bn = 640
    xm, xs, axm = _prep_call(x, w_msg_p, bm_p, w_self, b_self, am_p, bn)

    ex, d2 = _sc_alpha_call(src, dst, axm, aem, has_edge)
    da, db = d2[0], d2[1]

    se1 = 32 if hc >= 1024 else 128
    acc2 = _sc_out1_call(src, dst, ex, da, db, xm, h, cout, se1)

    if has_edge:
        se2 = 80 if h * D_EDGE > 128 else 128
        s2 = _sc_sedge_call(src, dst, ex, da, db, ea16, h, se2)
        wp = w_edge.reshape(D_EDGE, h, cout).transpose(1, 0, 2).reshape(h * D_EDGE, cout)
        out = _post_edge_call(acc2, s2, wp, xs, 640, apply_elu=not last)
    else:
        out = _post_plain_call(acc2, xs, 640)
    return out


def kernel(x, edge_index, edge_attr, w1_msg, b1_msg, w1_edge, b1_edge, a1,
           w1_self, b1_self, w2_msg, b2_msg, w2_edge, b2_edge, a2, w2_self,
           b2_self, w3_msg, b3_msg, w3_edge, b3_edge, a3, w3_self, b3_self,
           w4_msg, b4_msg, a4, w4_self, b4_self):
    src = edge_index[0]
    dst = edge_index[1]
    x = jnp.pad(x, ((0, NP - N), (0, 0)))
    ea16 = jnp.pad(edge_attr, ((0, 0), (0, HP - D_EDGE)))

    def _am(att, h, cout):
        hc = h * cout
        eye = jnp.eye(HP, dtype=f32)[:h]
        return (att[0][:, :, None] * eye[:, None, :]).reshape(hc, HP)

    v_cat = jnp.concatenate([
        jnp.pad(w1_edge @ _am(a1, 8, 128), ((0, HP - D_EDGE), (0, 0))),
        jnp.pad(w2_edge @ _am(a2, 8, 64), ((0, HP - D_EDGE), (0, 0))),
        jnp.pad(w3_edge @ _am(a3, 16, 32), ((0, HP - D_EDGE), (0, 0))),
    ], axis=1)
    aem_cat = _mm_call(ea16, v_cat, 3200)
    aem1 = aem_cat[:, 0:HP]
    aem2 = aem_cat[:, HP:2 * HP]
    aem3 = aem_cat[:, 2 * HP:3 * HP]

    h = _layer(x, src, dst, ea16, aem1, w1_msg, b1_msg, w1_edge, b1_edge, a1,
               w1_self, b1_self, 8, 128, last=False)
    h = _layer(h, src, dst, ea16, aem2, w2_msg, b2_msg, w2_edge, b2_edge, a2,
               w2_self, b2_self, 8, 64, last=False)
    h = _layer(h, src, dst, ea16, aem3, w3_msg, b3_msg, w3_edge, b3_edge, a3,
               w3_self, b3_self, 16, 32, last=False)
    z = _layer(h, src, dst, None, None, w4_msg, b4_msg, None, None, a4,
               w4_self, b4_self, 1, 2, last=True)
    return z[:N]
